# baseline (speedup 1.0000x reference)
import jax
import jax.numpy as jnp
from jax import lax
from jax.experimental import pallas as pl
from jax.experimental.pallas import tpu as pltpu
from jax.experimental.pallas import tpu_sc as plsc

D = 16
NC = 2
NS = 16
NW = NC * NS
CH = 3200


def _tc_table_body(x_ref, y_ref):
    x = x_ref[...]
    y_ref[...] = x.reshape(16, 128, 8).transpose(1, 2, 0).reshape(128, 128)


def _tc_out_body(x_ref, y_ref):
    x = x_ref[...]
    y_ref[...] = x.reshape(128, 25, 128).transpose(1, 2, 0).reshape(200, 16, 128)


def _sc_gather_body(table_hbm, idx_hbm, out_hbm, idx_v, r0, r1, gs0, gs1,
                    os0, os1):
    n_per_w = idx_hbm.shape[0] // NW
    n_chunks = n_per_w // CH
    wid = lax.axis_index("s") * NC + lax.axis_index("c")
    base = wid * n_per_w

    pltpu.sync_copy(idx_hbm.at[pl.ds(base, n_per_w)], idx_v)

    def g(k, rbuf, sem):
        pltpu.async_copy(table_hbm.at[idx_v.at[pl.ds(k * CH, CH)]], rbuf, sem)

    def gwait(rbuf, sem):
        pltpu.make_async_copy(table_hbm.at[idx_v.at[pl.ds(0, CH)]], rbuf,
                              sem).wait()

    def put(k, rbuf, sem):
        pltpu.async_copy(rbuf, out_hbm.at[pl.ds(base + k * CH, CH)], sem)

    def pwait(rbuf, sem):
        pltpu.make_async_copy(rbuf, out_hbm.at[pl.ds(0, CH)], sem).wait()

    g(0, r0, gs0)
    g(1, r1, gs1)

    def step(i, carry):
        k0 = 2 * i
        gwait(r0, gs0)
        put(k0, r0, os0)
        pwait(r0, os0)

        @pl.when(k0 + 2 < n_chunks)
        def _():
            g(k0 + 2, r0, gs0)

        gwait(r1, gs1)
        put(k0 + 1, r1, os1)
        pwait(r1, os1)

        @pl.when(k0 + 3 < n_chunks)
        def _():
            g(k0 + 3, r1, gs1)

        return carry

    lax.fori_loop(0, n_chunks // 2, step, 0)


def kernel(embedding, mb_feats):
    batch, hist = mb_feats.shape
    n_rows = embedding.shape[0]
    n_idx = batch * hist
    trows = n_rows * D // 128

    table_lin = pl.pallas_call(
        _tc_table_body,
        grid=((n_rows + 1023) // 1024,),
        in_specs=[pl.BlockSpec((D, 1024), lambda i: (0, i))],
        out_specs=pl.BlockSpec((128, 128), lambda i: (i, 0)),
        out_shape=jax.ShapeDtypeStruct((trows, 128), jnp.float32),
    )(embedding.T)

    mesh = plsc.VectorSubcoreMesh(core_axis_name="c", subcore_axis_name="s")
    rows = pl.kernel(
        _sc_gather_body,
        out_type=jax.ShapeDtypeStruct((n_idx, D), jnp.float32),
        mesh=mesh,
        scratch_types=[
            pltpu.VMEM((n_idx // NW,), jnp.int32),
            pltpu.VMEM((CH, D), jnp.float32),
            pltpu.VMEM((CH, D), jnp.float32),
            pltpu.SemaphoreType.DMA,
            pltpu.SemaphoreType.DMA,
            pltpu.SemaphoreType.DMA,
            pltpu.SemaphoreType.DMA,
        ],
        compiler_params=pltpu.CompilerParams(use_tc_tiling_on_sc=False),
    )(table_lin.reshape(n_rows, D), mb_feats.reshape(n_idx))

    hd = hist * D // 128
    out_t = pl.pallas_call(
        _tc_out_body,
        grid=(batch // 128,),
        in_specs=[pl.BlockSpec((128 * hd, 128), lambda i: (i, 0))],
        out_specs=pl.BlockSpec((hist, D, 128), lambda i: (0, 0, i)),
        out_shape=jax.ShapeDtypeStruct((hist, D, batch), jnp.float32),
    )(rows.reshape(batch * hd, 128))

    return lax.stop_gradient(out_t.transpose(2, 0, 1))


# speedup vs baseline: 1.3548x; 1.3548x over previous
import jax
import jax.numpy as jnp
from jax import lax
from jax.experimental import pallas as pl
from jax.experimental.pallas import tpu as pltpu
from jax.experimental.pallas import tpu_sc as plsc

D = 16
NC = 2
NS = 16
NW = NC * NS
CH = 3200


S = 124928
NFB = 976


def _tc_table_body(x0, x1, x2, x3, x4, x5, x6, x7, xt, y_ref):
    i = pl.program_id(0)
    xs = (x0, x1, x2, x3, x4, x5, x6, x7)

    @pl.when(i < NFB)
    def _():
        for r in range(8):
            y_ref[:, pl.ds(r * D, D)] = xs[r][...].T

    @pl.when(i == NFB)
    def _():
        x = xt[...]
        y_ref[...] = (x.reshape(16, 128, 8).transpose(1, 2, 0)
                      .reshape(128, 128))


def _tc_out_body(x_ref, y_ref):
    x = x_ref[...]
    y_ref[...] = x.reshape(128, 25, 128).transpose(1, 2, 0).reshape(200, 16, 128)


def _sc_gather_body(table_hbm, idx_hbm, out_hbm, idx_v, r0, r1, gs0, gs1,
                    os0, os1):
    n_per_w = idx_hbm.shape[0] // NW
    n_chunks = n_per_w // CH
    wid = lax.axis_index("s") * NC + lax.axis_index("c")
    base = wid * n_per_w

    pltpu.sync_copy(idx_hbm.at[pl.ds(base, n_per_w)], idx_v)

    def g(k, rbuf, sem):
        pltpu.async_copy(table_hbm.at[idx_v.at[pl.ds(k * CH, CH)]], rbuf, sem)

    def gwait(rbuf, sem):
        pltpu.make_async_copy(table_hbm.at[idx_v.at[pl.ds(0, CH)]], rbuf,
                              sem).wait()

    def put(k, rbuf, sem):
        pltpu.async_copy(rbuf, out_hbm.at[pl.ds(base + k * CH, CH)], sem)

    def pwait(rbuf, sem):
        pltpu.make_async_copy(rbuf, out_hbm.at[pl.ds(0, CH)], sem).wait()

    g(0, r0, gs0)
    g(1, r1, gs1)

    def step(i, carry):
        k0 = 2 * i
        gwait(r0, gs0)
        put(k0, r0, os0)
        pwait(r0, os0)

        @pl.when(k0 + 2 < n_chunks)
        def _():
            g(k0 + 2, r0, gs0)

        gwait(r1, gs1)
        put(k0 + 1, r1, os1)
        pwait(r1, os1)

        @pl.when(k0 + 3 < n_chunks)
        def _():
            g(k0 + 3, r1, gs1)

        return carry

    lax.fori_loop(0, n_chunks // 2, step, 0)


def kernel(embedding, mb_feats):
    batch, hist = mb_feats.shape
    n_rows = embedding.shape[0]
    n_idx = batch * hist
    trows = n_rows * D // 128
    i_flat = mb_feats.reshape(n_idx)
    idx_p = jnp.where(i_flat < 8 * S, (i_flat % S) * 8 + i_flat // S, i_flat)

    emb_t = embedding.T
    fast_specs = [
        pl.BlockSpec((D, 128), lambda i, r=r: (0, jnp.minimum(i, NFB - 1)
                                               + r * NFB))
        for r in range(8)
    ]
    tail_spec = pl.BlockSpec((D, 1024), lambda i: (0, NFB))
    table_lin = pl.pallas_call(
        _tc_table_body,
        grid=(NFB + 1,),
        in_specs=fast_specs + [tail_spec],
        out_specs=pl.BlockSpec((128, 128), lambda i: (i, 0)),
        out_shape=jax.ShapeDtypeStruct((trows, 128), jnp.float32),
    )(*([emb_t] * 9))

    mesh = plsc.VectorSubcoreMesh(core_axis_name="c", subcore_axis_name="s")
    rows = pl.kernel(
        _sc_gather_body,
        out_type=jax.ShapeDtypeStruct((n_idx, D), jnp.float32),
        mesh=mesh,
        scratch_types=[
            pltpu.VMEM((n_idx // NW,), jnp.int32),
            pltpu.VMEM((CH, D), jnp.float32),
            pltpu.VMEM((CH, D), jnp.float32),
            pltpu.SemaphoreType.DMA,
            pltpu.SemaphoreType.DMA,
            pltpu.SemaphoreType.DMA,
            pltpu.SemaphoreType.DMA,
        ],
        compiler_params=pltpu.CompilerParams(use_tc_tiling_on_sc=False),
    )(table_lin.reshape(n_rows, D), idx_p)

    hd = hist * D // 128
    out_t = pl.pallas_call(
        _tc_out_body,
        grid=(batch // 128,),
        in_specs=[pl.BlockSpec((128 * hd, 128), lambda i: (i, 0))],
        out_specs=pl.BlockSpec((hist, D, 128), lambda i: (0, 0, i)),
        out_shape=jax.ShapeDtypeStruct((hist, D, batch), jnp.float32),
    )(rows.reshape(batch * hd, 128))

    return lax.stop_gradient(out_t.transpose(2, 0, 1))


# speedup vs baseline: 2.7463x; 2.0271x over previous
import jax
import jax.numpy as jnp
from jax import lax
from jax.experimental import pallas as pl
from jax.experimental.pallas import tpu as pltpu
from jax.experimental.pallas import tpu_sc as plsc

D = 16
NC = 2
NS = 16
NW = NC * NS
CH = 3200


S = 124928
NFB = 61
TBW = 2048


def _tc_table_body(x0, x1, x2, x3, x4, x5, x6, x7, xt, y_ref):
    i = pl.program_id(0)
    xs = (x0, x1, x2, x3, x4, x5, x6, x7)

    @pl.when(i < NFB)
    def _():
        for r in range(8):
            y_ref[:, pl.ds(r * D, D)] = xs[r][...].T

    @pl.when(i == NFB)
    def _():
        x = xt[...]
        y_ref[pl.ds(0, TBW // 8), :] = (
            x.reshape(16, TBW // 8, 8).transpose(1, 2, 0)
            .reshape(TBW // 8, 128))


def _tc_out_body(x_ref, y_ref):
    y_ref[...] = x_ref[...].T.reshape(8, D, 4096)


def _sc_gather_body(table_hbm, idx_hbm, out_hbm, idx_v, r0, r1, gs0, gs1,
                    os0, os1):
    n_per_w = idx_hbm.shape[0] // NW
    n_chunks = n_per_w // CH
    wid = lax.axis_index("s") * NC + lax.axis_index("c")
    base = wid * n_per_w

    pltpu.sync_copy(idx_hbm.at[pl.ds(base, n_per_w)], idx_v)

    def g(k, rbuf, sem):
        pltpu.async_copy(table_hbm.at[idx_v.at[pl.ds(k * CH, CH)]], rbuf, sem)

    def gwait(rbuf, sem):
        pltpu.make_async_copy(table_hbm.at[idx_v.at[pl.ds(0, CH)]], rbuf,
                              sem).wait()

    def put(k, rbuf, sem):
        pltpu.async_copy(rbuf, out_hbm.at[pl.ds(base + k * CH, CH)], sem)

    def pwait(rbuf, sem):
        pltpu.make_async_copy(rbuf, out_hbm.at[pl.ds(0, CH)], sem).wait()

    g(0, r0, gs0)
    g(1, r1, gs1)

    def step(i, carry):
        k0 = 2 * i
        gwait(r0, gs0)
        put(k0, r0, os0)
        pwait(r0, os0)

        @pl.when(k0 + 2 < n_chunks)
        def _():
            g(k0 + 2, r0, gs0)

        gwait(r1, gs1)
        put(k0 + 1, r1, os1)
        pwait(r1, os1)

        @pl.when(k0 + 3 < n_chunks)
        def _():
            g(k0 + 3, r1, gs1)

        return carry

    lax.fori_loop(0, n_chunks // 2, step, 0)


def kernel(embedding, mb_feats):
    batch, hist = mb_feats.shape
    n_rows = embedding.shape[0]
    n_idx = batch * hist
    trows = n_rows * D // 128
    i_flat = (mb_feats.reshape(batch, hist // 8, 8).transpose(1, 0, 2)
              .reshape(n_idx))
    idx_p = jnp.where(i_flat < 8 * S, (i_flat % S) * 8 + i_flat // S, i_flat)

    emb_t = embedding.T
    fast_specs = [
        pl.BlockSpec((D, TBW), lambda i, r=r: (0, jnp.minimum(i, NFB - 1)
                                               + r * NFB))
        for r in range(8)
    ]
    tail_spec = pl.BlockSpec((D, TBW), lambda i: (0, 8 * NFB))
    table_lin = pl.pallas_call(
        _tc_table_body,
        grid=(NFB + 1,),
        in_specs=fast_specs + [tail_spec],
        out_specs=pl.BlockSpec((TBW, 128), lambda i: (i, 0)),
        out_shape=jax.ShapeDtypeStruct((trows, 128), jnp.float32),
    )(*([emb_t] * 9))

    mesh = plsc.VectorSubcoreMesh(core_axis_name="c", subcore_axis_name="s")
    rows = pl.kernel(
        _sc_gather_body,
        out_type=jax.ShapeDtypeStruct((n_idx, D), jnp.float32),
        mesh=mesh,
        scratch_types=[
            pltpu.VMEM((n_idx // NW,), jnp.int32),
            pltpu.VMEM((CH, D), jnp.float32),
            pltpu.VMEM((CH, D), jnp.float32),
            pltpu.SemaphoreType.DMA,
            pltpu.SemaphoreType.DMA,
            pltpu.SemaphoreType.DMA,
            pltpu.SemaphoreType.DMA,
        ],
        compiler_params=pltpu.CompilerParams(use_tc_tiling_on_sc=False),
    )(table_lin.reshape(n_rows, D), idx_p)

    out_t = pl.pallas_call(
        _tc_out_body,
        grid=(hist // 8,),
        in_specs=[pl.BlockSpec((batch, 128), lambda i: (i, 0))],
        out_specs=pl.BlockSpec((8, D, batch), lambda i: (i, 0, 0)),
        out_shape=jax.ShapeDtypeStruct((hist, D, batch), jnp.float32),
    )(rows.reshape(n_idx * D // 128, 128))

    return lax.stop_gradient(out_t.transpose(2, 0, 1))
